# Initial kernel scaffold; baseline (speedup 1.0000x reference)
import functools

import jax
import jax.numpy as jnp
from jax import lax
from jax.experimental import pallas as pl
from jax.experimental.pallas import tpu as pltpu
from jax.experimental.pallas import tpu_sc as plsc

N = 10000
E = 320000
D_IN = 128
D_H = 64
D_OUT = 32

NC = 2
NS = 16
NW = NC * NS

C = 128
CH = (E // NW + C - 1) // C
EPW = CH * C
E_PAD = NW * EPW
ACC_N = ((N + NS) // NS) * NS
RPT = ACC_N // NS
DEG_W = 16


def _sc_mesh():
    return plsc.VectorSubcoreMesh(core_axis_name="c", subcore_axis_name="s")


def _make_deg_kernel():
    @functools.partial(
        pl.kernel,
        mesh=_sc_mesh(),
        out_type=jax.ShapeDtypeStruct((NC * ACC_N, DEG_W), jnp.float32),
        scratch_types=[
            pltpu.VMEM((CH, C), jnp.int32),
            pltpu.VMEM((C, DEG_W), jnp.float32),
            pltpu.VMEM_SHARED((ACC_N, DEG_W), jnp.float32),
        ],
    )
    def deg_kernel(dst2d, ones_hbm, zeros_hbm, out_hbm, dst_v, ones_v, acc_sh):
        cid = lax.axis_index("c")
        sid = lax.axis_index("s")
        wid = cid * NS + sid
        pltpu.sync_copy(dst2d.at[pl.ds(wid * CH, CH)], dst_v)
        pltpu.sync_copy(ones_hbm, ones_v)
        pltpu.sync_copy(zeros_hbm.at[pl.ds(sid * RPT, RPT)],
                        acc_sh.at[pl.ds(sid * RPT, RPT)])
        plsc.subcore_barrier()

        def body(j, carry):
            pltpu.sync_copy(ones_v, acc_sh.at[dst_v.at[j]], add=True)
            return carry

        lax.fori_loop(0, CH, body, 0)
        plsc.subcore_barrier()
        pltpu.sync_copy(acc_sh.at[pl.ds(sid * RPT, RPT)],
                        out_hbm.at[pl.ds(cid * ACC_N + sid * RPT, RPT)])

    return deg_kernel


def _make_seg_kernel(D):
    @functools.partial(
        pl.kernel,
        mesh=_sc_mesh(),
        out_type=jax.ShapeDtypeStruct((NC * ACC_N, D), jnp.float32),
        scratch_types=[
            pltpu.VMEM((CH, C), jnp.int32),
            pltpu.VMEM((CH, C), jnp.int32),
            pltpu.VMEM((C, D), jnp.float32),
            pltpu.VMEM_SHARED((ACC_N, D), jnp.float32),
            pltpu.SemaphoreType.DMA,
        ],
    )
    def seg_kernel(y_hbm, src2d, dst2d, zeros_hbm, out_hbm,
                   src_v, dst_v, rows_v, acc_sh, sem):
        cid = lax.axis_index("c")
        sid = lax.axis_index("s")
        wid = cid * NS + sid
        pltpu.sync_copy(src2d.at[pl.ds(wid * CH, CH)], src_v)
        pltpu.sync_copy(dst2d.at[pl.ds(wid * CH, CH)], dst_v)
        pltpu.sync_copy(zeros_hbm.at[pl.ds(sid * RPT, RPT)],
                        acc_sh.at[pl.ds(sid * RPT, RPT)])
        plsc.subcore_barrier()

        def body(j, carry):
            pltpu.async_copy(y_hbm.at[src_v.at[j]], rows_v, sem).wait()
            pltpu.sync_copy(rows_v, acc_sh.at[dst_v.at[j]], add=True)
            return carry

        lax.fori_loop(0, CH, body, 0)
        plsc.subcore_barrier()
        pltpu.sync_copy(acc_sh.at[pl.ds(sid * RPT, RPT)],
                        out_hbm.at[pl.ds(cid * ACC_N + sid * RPT, RPT)])

    return seg_kernel


_deg_kernel = _make_deg_kernel()
_seg_kernel_h = _make_seg_kernel(D_H)
_seg_kernel_o = _make_seg_kernel(D_OUT)


BN = 500


def _tc_a_body(x_ref, w_ref, d0_ref, d1_ref, y_ref, dis_ref):
    deg = d0_ref[...] + d1_ref[...] + 1.0
    dis = lax.rsqrt(deg)
    dis_ref[...] = dis
    xw = jnp.dot(x_ref[...], w_ref[...], preferred_element_type=jnp.float32)
    y_ref[...] = dis * xw


def _tc_b_body(a0_ref, a1_ref, y1_ref, dis_ref, w_ref, b_ref, y2_ref):
    dis = dis_ref[...]
    h = dis * (a0_ref[...] + a1_ref[...] + y1_ref[...]) + b_ref[...]
    h = jnp.maximum(h, 0.0)
    y2_ref[...] = dis * jnp.dot(h, w_ref[...], preferred_element_type=jnp.float32)


def _tc_c_body(a0_ref, a1_ref, y2_ref, dis_ref, b_ref, out_ref):
    out_ref[...] = (dis_ref[...] * (a0_ref[...] + a1_ref[...] + y2_ref[...])
                    + b_ref[...])


def _row_spec(d):
    return pl.BlockSpec((BN, d), lambda i: (i, 0))


def _full_spec(shape):
    return pl.BlockSpec(shape, lambda i: (0, 0))


def _tc_a(x, W1, deg0, deg1):
    return pl.pallas_call(
        _tc_a_body,
        grid=(N // BN,),
        in_specs=[_row_spec(D_IN), _full_spec((D_IN, D_H)),
                  _row_spec(1), _row_spec(1)],
        out_specs=[_row_spec(D_H), _row_spec(1)],
        out_shape=[jax.ShapeDtypeStruct((N, D_H), jnp.float32),
                   jax.ShapeDtypeStruct((N, 1), jnp.float32)],
    )(x, W1, deg0, deg1)


def _tc_b(a0, a1, y1, dis, W2, b1):
    return pl.pallas_call(
        _tc_b_body,
        grid=(N // BN,),
        in_specs=[_row_spec(D_H), _row_spec(D_H), _row_spec(D_H), _row_spec(1),
                  _full_spec((D_H, D_OUT)), _full_spec((1, D_H))],
        out_specs=_row_spec(D_OUT),
        out_shape=jax.ShapeDtypeStruct((N, D_OUT), jnp.float32),
    )(a0, a1, y1, dis, W2, b1)


def _tc_c(a0, a1, y2, dis, b2):
    return pl.pallas_call(
        _tc_c_body,
        grid=(N // BN,),
        in_specs=[_row_spec(D_OUT), _row_spec(D_OUT), _row_spec(D_OUT),
                  _row_spec(1), _full_spec((1, D_OUT))],
        out_specs=_row_spec(D_OUT),
        out_shape=jax.ShapeDtypeStruct((N, D_OUT), jnp.float32),
    )(a0, a1, y2, dis, b2)


def kernel(x, edge_index, W1, b1, W2, b2):
    src = edge_index[0]
    dst = edge_index[1]
    pad = E_PAD - E
    sink = jnp.full((pad,), N, dtype=jnp.int32)
    src2d = jnp.concatenate([src, sink]).reshape(NW * CH, C)
    dst2d = jnp.concatenate([dst, sink]).reshape(NW * CH, C)

    ones_rows = jnp.ones((C, DEG_W), jnp.float32)
    zeros64 = jnp.zeros((ACC_N, D_H), jnp.float32)

    degp = _deg_kernel(dst2d, ones_rows, zeros64[:, :DEG_W])
    deg0 = degp[:N, 0:1]
    deg1 = degp[ACC_N:ACC_N + N, 0:1]

    y1, dis = _tc_a(x, W1, deg0, deg1)
    y1_pad = jnp.concatenate([y1, jnp.zeros((ACC_N - N, D_H), jnp.float32)])

    acc1 = _seg_kernel_h(y1_pad, src2d, dst2d, zeros64)
    a10 = acc1[:N]
    a11 = acc1[ACC_N:ACC_N + N]

    y2 = _tc_b(a10, a11, y1, dis, W2, b1.reshape(1, D_H))
    y2_pad = jnp.concatenate([y2, jnp.zeros((ACC_N - N, D_OUT), jnp.float32)])

    acc2 = _seg_kernel_o(y2_pad, src2d, dst2d, zeros64[:, :D_OUT])
    a20 = acc2[:N]
    a21 = acc2[ACC_N:ACC_N + N]

    return _tc_c(a20, a21, y2, dis, b2.reshape(1, D_OUT))


# speedup vs baseline: 15.7612x; 15.7612x over previous
import functools

import jax
import jax.numpy as jnp
from jax import lax
from jax.experimental import pallas as pl
from jax.experimental.pallas import tpu as pltpu
from jax.experimental.pallas import tpu_sc as plsc

N = 10000
E = 320000
D_IN = 128
D_H = 64
D_OUT = 32

NC = 2
NS = 16
NW = NC * NS

C = 128
CH = 80
EPW = CH * C
E_PAD = NW * EPW
ACC_N = 10112
RPT = ACC_N // NS
DEG_W = 16


def _sc_mesh():
    return plsc.VectorSubcoreMesh(core_axis_name="c", subcore_axis_name="s")


def _make_deg_kernel():
    @functools.partial(
        pl.kernel,
        mesh=_sc_mesh(),
        compiler_params=pltpu.CompilerParams(use_tc_tiling_on_sc=False),
        out_type=jax.ShapeDtypeStruct((NC * ACC_N, DEG_W), jnp.float32),
        scratch_types=[
            pltpu.VMEM((CH, C), jnp.int32),
            pltpu.VMEM((C, DEG_W), jnp.float32),
            pltpu.VMEM_SHARED((ACC_N, DEG_W), jnp.float32),
        ],
    )
    def deg_kernel(dst2d, ones_hbm, zeros_hbm, out_hbm, dst_v, ones_v, acc_sh):
        cid = lax.axis_index("c")
        sid = lax.axis_index("s")
        wid = cid * NS + sid
        pltpu.sync_copy(dst2d.at[pl.ds(wid * CH, CH)], dst_v)
        pltpu.sync_copy(ones_hbm, ones_v)
        pltpu.sync_copy(zeros_hbm.at[pl.ds(sid * RPT, RPT)],
                        acc_sh.at[pl.ds(sid * RPT, RPT)])
        plsc.subcore_barrier()

        def body(j, carry):
            pltpu.sync_copy(ones_v, acc_sh.at[dst_v.at[j]], add=True)
            return carry

        lax.fori_loop(0, CH, body, 0)
        plsc.subcore_barrier()
        pltpu.sync_copy(acc_sh.at[pl.ds(sid * RPT, RPT)],
                        out_hbm.at[pl.ds(cid * ACC_N + sid * RPT, RPT)])

    return deg_kernel


def _make_seg_kernel(D):
    @functools.partial(
        pl.kernel,
        mesh=_sc_mesh(),
        compiler_params=pltpu.CompilerParams(use_tc_tiling_on_sc=False),
        out_type=jax.ShapeDtypeStruct((NC * ACC_N, D), jnp.float32),
        scratch_types=[
            pltpu.VMEM((CH, C), jnp.int32),
            pltpu.VMEM((CH, C), jnp.int32),
            pltpu.VMEM((C, D), jnp.float32),
            pltpu.VMEM_SHARED((ACC_N, D), jnp.float32),
            pltpu.SemaphoreType.DMA,
        ],
    )
    def seg_kernel(y_hbm, src2d, dst2d, zeros_hbm, out_hbm,
                   src_v, dst_v, rows_v, acc_sh, sem):
        cid = lax.axis_index("c")
        sid = lax.axis_index("s")
        wid = cid * NS + sid
        pltpu.sync_copy(src2d.at[pl.ds(wid * CH, CH)], src_v)
        pltpu.sync_copy(dst2d.at[pl.ds(wid * CH, CH)], dst_v)
        pltpu.sync_copy(zeros_hbm.at[pl.ds(sid * RPT, RPT)],
                        acc_sh.at[pl.ds(sid * RPT, RPT)])
        plsc.subcore_barrier()

        def body(j, carry):
            pltpu.async_copy(y_hbm.at[src_v.at[j]], rows_v, sem).wait()
            pltpu.sync_copy(rows_v, acc_sh.at[dst_v.at[j]], add=True)
            return carry

        lax.fori_loop(0, CH, body, 0)
        plsc.subcore_barrier()
        pltpu.sync_copy(acc_sh.at[pl.ds(sid * RPT, RPT)],
                        out_hbm.at[pl.ds(cid * ACC_N + sid * RPT, RPT)])

    return seg_kernel


_deg_kernel = _make_deg_kernel()
_seg_kernel_h = _make_seg_kernel(D_H)
_seg_kernel_o = _make_seg_kernel(D_OUT)


BN = 1000


def _tc_a_body(x_ref, w_ref, d0_ref, d1_ref, y_ref, dis_ref):
    deg = d0_ref[...] + d1_ref[...] + 1.0
    dis = lax.rsqrt(deg)
    dis_ref[...] = dis
    xw = jnp.dot(x_ref[...], w_ref[...], preferred_element_type=jnp.float32)
    y_ref[...] = dis * xw


def _tc_b_body(a0_ref, a1_ref, y1_ref, dis_ref, w_ref, b_ref, y2_ref):
    dis = dis_ref[...]
    h = dis * (a0_ref[...] + a1_ref[...] + y1_ref[...]) + b_ref[...]
    h = jnp.maximum(h, 0.0)
    y2_ref[...] = dis * jnp.dot(h, w_ref[...], preferred_element_type=jnp.float32)


def _tc_c_body(a0_ref, a1_ref, y2_ref, dis_ref, b_ref, out_ref):
    out_ref[...] = (dis_ref[...] * (a0_ref[...] + a1_ref[...] + y2_ref[...])
                    + b_ref[...])


def _row_spec(d):
    return pl.BlockSpec((BN, d), lambda i: (i, 0))


def _full_spec(shape):
    return pl.BlockSpec(shape, lambda i: (0, 0))


def _tc_a(x, W1, deg0, deg1):
    return pl.pallas_call(
        _tc_a_body,
        grid=(N // BN,),
        in_specs=[_row_spec(D_IN), _full_spec((D_IN, D_H)),
                  _row_spec(1), _row_spec(1)],
        out_specs=[_row_spec(D_H), _row_spec(1)],
        out_shape=[jax.ShapeDtypeStruct((N, D_H), jnp.float32),
                   jax.ShapeDtypeStruct((N, 1), jnp.float32)],
    )(x, W1, deg0, deg1)


def _tc_b(a0, a1, y1, dis, W2, b1):
    return pl.pallas_call(
        _tc_b_body,
        grid=(N // BN,),
        in_specs=[_row_spec(D_H), _row_spec(D_H), _row_spec(D_H), _row_spec(1),
                  _full_spec((D_H, D_OUT)), _full_spec((1, D_H))],
        out_specs=_row_spec(D_OUT),
        out_shape=jax.ShapeDtypeStruct((N, D_OUT), jnp.float32),
    )(a0, a1, y1, dis, W2, b1)


def _tc_c(a0, a1, y2, dis, b2):
    return pl.pallas_call(
        _tc_c_body,
        grid=(N // BN,),
        in_specs=[_row_spec(D_OUT), _row_spec(D_OUT), _row_spec(D_OUT),
                  _row_spec(1), _full_spec((1, D_OUT))],
        out_specs=_row_spec(D_OUT),
        out_shape=jax.ShapeDtypeStruct((N, D_OUT), jnp.float32),
    )(a0, a1, y2, dis, b2)


def kernel(x, edge_index, W1, b1, W2, b2):
    src = edge_index[0]
    dst = edge_index[1]
    pad = E_PAD - E
    sink = jnp.full((pad,), N, dtype=jnp.int32)
    src2d = jnp.concatenate([src, sink]).reshape(NW * CH, C)
    dst2d = jnp.concatenate([dst, sink]).reshape(NW * CH, C)

    ones_rows = jnp.ones((C, DEG_W), jnp.float32)
    zeros64 = jnp.zeros((ACC_N, D_H), jnp.float32)

    degp = _deg_kernel(dst2d, ones_rows, zeros64[:, :DEG_W])
    deg0 = degp[:N, 0:1]
    deg1 = degp[ACC_N:ACC_N + N, 0:1]

    y1, dis = _tc_a(x, W1, deg0, deg1)
    y1_pad = jnp.concatenate([y1, jnp.zeros((ACC_N - N, D_H), jnp.float32)])

    acc1 = _seg_kernel_h(y1_pad, src2d, dst2d, zeros64)
    a10 = acc1[:N]
    a11 = acc1[ACC_N:ACC_N + N]

    y2 = _tc_b(a10, a11, y1, dis, W2, b1.reshape(1, D_H))
    y2_pad = jnp.concatenate([y2, jnp.zeros((ACC_N - N, D_OUT), jnp.float32)])

    acc2 = _seg_kernel_o(y2_pad, src2d, dst2d, zeros64[:, :D_OUT])
    a20 = acc2[:N]
    a21 = acc2[ACC_N:ACC_N + N]

    return _tc_c(a20, a21, y2, dis, b2.reshape(1, D_OUT))


# speedup vs baseline: 17.6983x; 1.1229x over previous
import functools

import jax
import jax.numpy as jnp
from jax import lax
from jax.experimental import pallas as pl
from jax.experimental.pallas import tpu as pltpu
from jax.experimental.pallas import tpu_sc as plsc

N = 10000
E = 320000
D_IN = 128
D_H = 64
D_OUT = 32

NC = 2
NS = 16
NW = NC * NS

C = 128
CH = 80
EPW = CH * C
E_PAD = NW * EPW
ACC_N = 10112
RPT = ACC_N // NS
DEG_W = 16


def _sc_mesh():
    return plsc.VectorSubcoreMesh(core_axis_name="c", subcore_axis_name="s")


def _make_deg_kernel():
    @functools.partial(
        pl.kernel,
        mesh=_sc_mesh(),
        compiler_params=pltpu.CompilerParams(use_tc_tiling_on_sc=False),
        out_type=jax.ShapeDtypeStruct((NC * ACC_N, DEG_W), jnp.float32),
        scratch_types=[
            pltpu.VMEM((CH, C), jnp.int32),
            pltpu.VMEM((C, DEG_W), jnp.float32),
            pltpu.VMEM_SHARED((ACC_N, DEG_W), jnp.float32),
            pltpu.SemaphoreType.DMA,
            pltpu.SemaphoreType.DMA,
        ],
    )
    def deg_kernel(dst2d, ones_hbm, zeros_hbm, out_hbm, dst_v, ones_v, acc_sh,
                   s0, s1):
        cid = lax.axis_index("c")
        sid = lax.axis_index("s")
        wid = cid * NS + sid
        pltpu.sync_copy(dst2d.at[pl.ds(wid * CH, CH)], dst_v)
        pltpu.sync_copy(ones_hbm, ones_v)
        pltpu.sync_copy(zeros_hbm.at[pl.ds(sid * RPT, RPT)],
                        acc_sh.at[pl.ds(sid * RPT, RPT)])
        plsc.subcore_barrier()

        def s_start(j, sem):
            pltpu.async_copy(ones_v, acc_sh.at[dst_v.at[j]], sem, add=True)

        def s_wait(j, sem):
            pltpu.make_async_copy(ones_v, acc_sh.at[dst_v.at[j]], sem).wait()

        s_start(0, s0)
        s_start(1, s1)

        def body(i, carry):
            j0 = 2 * i
            s_wait(j0, s0)
            s_start(j0 + 2, s0)
            s_wait(j0 + 1, s1)
            s_start(j0 + 3, s1)
            return carry

        lax.fori_loop(0, CH // 2 - 1, body, 0)
        s_wait(CH - 2, s0)
        s_wait(CH - 1, s1)
        plsc.subcore_barrier()
        pltpu.sync_copy(acc_sh.at[pl.ds(sid * RPT, RPT)],
                        out_hbm.at[pl.ds(cid * ACC_N + sid * RPT, RPT)])

    return deg_kernel


def _make_seg_kernel(D):
    @functools.partial(
        pl.kernel,
        mesh=_sc_mesh(),
        compiler_params=pltpu.CompilerParams(use_tc_tiling_on_sc=False),
        out_type=jax.ShapeDtypeStruct((NC * ACC_N, D), jnp.float32),
        scratch_types=[
            pltpu.VMEM((CH, C), jnp.int32),
            pltpu.VMEM((CH, C), jnp.int32),
            pltpu.VMEM((C, D), jnp.float32),
            pltpu.VMEM((C, D), jnp.float32),
            pltpu.VMEM_SHARED((ACC_N, D), jnp.float32),
            pltpu.SemaphoreType.DMA,
            pltpu.SemaphoreType.DMA,
            pltpu.SemaphoreType.DMA,
            pltpu.SemaphoreType.DMA,
        ],
    )
    def seg_kernel(y_hbm, src2d, dst2d, zeros_hbm, out_hbm,
                   src_v, dst_v, rows0, rows1, acc_sh, g0, g1, s0, s1):
        cid = lax.axis_index("c")
        sid = lax.axis_index("s")
        wid = cid * NS + sid
        pltpu.sync_copy(src2d.at[pl.ds(wid * CH, CH)], src_v)
        pltpu.sync_copy(dst2d.at[pl.ds(wid * CH, CH)], dst_v)
        pltpu.sync_copy(zeros_hbm.at[pl.ds(sid * RPT, RPT)],
                        acc_sh.at[pl.ds(sid * RPT, RPT)])
        plsc.subcore_barrier()

        def g_start(j, buf, sem):
            pltpu.async_copy(y_hbm.at[src_v.at[j]], buf, sem)

        def g_wait(j, buf, sem):
            pltpu.make_async_copy(y_hbm.at[src_v.at[j]], buf, sem).wait()

        def s_start(j, buf, sem):
            pltpu.async_copy(buf, acc_sh.at[dst_v.at[j]], sem, add=True)

        def s_wait(j, buf, sem):
            pltpu.make_async_copy(buf, acc_sh.at[dst_v.at[j]], sem).wait()

        g_start(0, rows0, g0)
        g_start(1, rows1, g1)

        def body(i, carry):
            j0 = 2 * i
            g_wait(j0, rows0, g0)
            s_start(j0, rows0, s0)
            g_wait(j0 + 1, rows1, g1)
            s_start(j0 + 1, rows1, s1)
            s_wait(j0, rows0, s0)
            g_start(j0 + 2, rows0, g0)
            s_wait(j0 + 1, rows1, s1)
            g_start(j0 + 3, rows1, g1)
            return carry

        lax.fori_loop(0, CH // 2 - 1, body, 0)
        j0 = CH - 2
        g_wait(j0, rows0, g0)
        s_start(j0, rows0, s0)
        g_wait(j0 + 1, rows1, g1)
        s_start(j0 + 1, rows1, s1)
        s_wait(j0, rows0, s0)
        s_wait(j0 + 1, rows1, s1)
        plsc.subcore_barrier()
        pltpu.sync_copy(acc_sh.at[pl.ds(sid * RPT, RPT)],
                        out_hbm.at[pl.ds(cid * ACC_N + sid * RPT, RPT)])

    return seg_kernel


_deg_kernel = _make_deg_kernel()
_seg_kernel_h = _make_seg_kernel(D_H)
_seg_kernel_o = _make_seg_kernel(D_OUT)


BN = 1000


def _tc_a_body(x_ref, w_ref, d0_ref, d1_ref, y_ref, dis_ref):
    deg = d0_ref[...] + d1_ref[...] + 1.0
    dis = lax.rsqrt(deg)
    dis_ref[...] = dis
    xw = jnp.dot(x_ref[...], w_ref[...], preferred_element_type=jnp.float32)
    y_ref[...] = dis * xw


def _tc_b_body(a0_ref, a1_ref, y1_ref, dis_ref, w_ref, b_ref, y2_ref):
    dis = dis_ref[...]
    h = dis * (a0_ref[...] + a1_ref[...] + y1_ref[...]) + b_ref[...]
    h = jnp.maximum(h, 0.0)
    y2_ref[...] = dis * jnp.dot(h, w_ref[...], preferred_element_type=jnp.float32)


def _tc_c_body(a0_ref, a1_ref, y2_ref, dis_ref, b_ref, out_ref):
    out_ref[...] = (dis_ref[...] * (a0_ref[...] + a1_ref[...] + y2_ref[...])
                    + b_ref[...])


def _row_spec(d):
    return pl.BlockSpec((BN, d), lambda i: (i, 0))


def _full_spec(shape):
    return pl.BlockSpec(shape, lambda i: (0, 0))


def _tc_a(x, W1, deg0, deg1):
    return pl.pallas_call(
        _tc_a_body,
        grid=(N // BN,),
        in_specs=[_row_spec(D_IN), _full_spec((D_IN, D_H)),
                  _row_spec(1), _row_spec(1)],
        out_specs=[_row_spec(D_H), _row_spec(1)],
        out_shape=[jax.ShapeDtypeStruct((N, D_H), jnp.float32),
                   jax.ShapeDtypeStruct((N, 1), jnp.float32)],
    )(x, W1, deg0, deg1)


def _tc_b(a0, a1, y1, dis, W2, b1):
    return pl.pallas_call(
        _tc_b_body,
        grid=(N // BN,),
        in_specs=[_row_spec(D_H), _row_spec(D_H), _row_spec(D_H), _row_spec(1),
                  _full_spec((D_H, D_OUT)), _full_spec((1, D_H))],
        out_specs=_row_spec(D_OUT),
        out_shape=jax.ShapeDtypeStruct((N, D_OUT), jnp.float32),
    )(a0, a1, y1, dis, W2, b1)


def _tc_c(a0, a1, y2, dis, b2):
    return pl.pallas_call(
        _tc_c_body,
        grid=(N // BN,),
        in_specs=[_row_spec(D_OUT), _row_spec(D_OUT), _row_spec(D_OUT),
                  _row_spec(1), _full_spec((1, D_OUT))],
        out_specs=_row_spec(D_OUT),
        out_shape=jax.ShapeDtypeStruct((N, D_OUT), jnp.float32),
    )(a0, a1, y2, dis, b2)


def kernel(x, edge_index, W1, b1, W2, b2):
    src = edge_index[0]
    dst = edge_index[1]
    pad = E_PAD - E
    sink = jnp.full((pad,), N, dtype=jnp.int32)
    src2d = jnp.concatenate([src, sink]).reshape(NW * CH, C)
    dst2d = jnp.concatenate([dst, sink]).reshape(NW * CH, C)

    ones_rows = jnp.ones((C, DEG_W), jnp.float32)
    zeros64 = jnp.zeros((ACC_N, D_H), jnp.float32)

    degp = _deg_kernel(dst2d, ones_rows, zeros64[:, :DEG_W])
    deg0 = degp[:N, 0:1]
    deg1 = degp[ACC_N:ACC_N + N, 0:1]

    y1, dis = _tc_a(x, W1, deg0, deg1)
    y1_pad = jnp.concatenate([y1, jnp.zeros((ACC_N - N, D_H), jnp.float32)])

    acc1 = _seg_kernel_h(y1_pad, src2d, dst2d, zeros64)
    a10 = acc1[:N]
    a11 = acc1[ACC_N:ACC_N + N]

    y2 = _tc_b(a10, a11, y1, dis, W2, b1.reshape(1, D_H))
    y2_pad = jnp.concatenate([y2, jnp.zeros((ACC_N - N, D_OUT), jnp.float32)])

    acc2 = _seg_kernel_o(y2_pad, src2d, dst2d, zeros64[:, :D_OUT])
    a20 = acc2[:N]
    a21 = acc2[ACC_N:ACC_N + N]

    return _tc_c(a20, a21, y2, dis, b2.reshape(1, D_OUT))


# speedup vs baseline: 33.9015x; 1.9155x over previous
import functools

import jax
import jax.numpy as jnp
from jax import lax
from jax.experimental import pallas as pl
from jax.experimental.pallas import tpu as pltpu
from jax.experimental.pallas import tpu_sc as plsc

N = 10000
E = 320000
D_IN = 128
D_H = 64
D_OUT = 32

NC = 2
NS = 16
NW = NC * NS

C = 128
CH = 80
EPW = CH * C
E_PAD = NW * EPW
ACC_N = 10112
RPT = ACC_N // NS
DEG_W = 16


def _sc_mesh():
    return plsc.VectorSubcoreMesh(core_axis_name="c", subcore_axis_name="s")


def _make_deg_kernel():
    @functools.partial(
        pl.kernel,
        mesh=_sc_mesh(),
        compiler_params=pltpu.CompilerParams(use_tc_tiling_on_sc=False),
        out_type=jax.ShapeDtypeStruct((NC * ACC_N, DEG_W), jnp.float32),
        scratch_types=[
            pltpu.VMEM((CH, C), jnp.int32),
            pltpu.VMEM((C, DEG_W), jnp.float32),
            pltpu.VMEM_SHARED((ACC_N, DEG_W), jnp.float32),
            pltpu.SemaphoreType.DMA,
            pltpu.SemaphoreType.DMA,
        ],
    )
    def deg_kernel(dst2d, ones_hbm, zeros_hbm, out_hbm, dst_v, ones_v, acc_sh,
                   s0, s1):
        cid = lax.axis_index("c")
        sid = lax.axis_index("s")
        wid = cid * NS + sid
        pltpu.sync_copy(dst2d.at[pl.ds(wid * CH, CH)], dst_v)
        pltpu.sync_copy(ones_hbm, ones_v)
        pltpu.sync_copy(zeros_hbm.at[pl.ds(sid * RPT, RPT)],
                        acc_sh.at[pl.ds(sid * RPT, RPT)])
        plsc.subcore_barrier()

        def s_start(j, sem):
            pltpu.async_copy(ones_v, acc_sh.at[dst_v.at[j]], sem, add=True)

        def s_wait(j, sem):
            pltpu.make_async_copy(ones_v, acc_sh.at[dst_v.at[j]], sem).wait()

        s_start(0, s0)
        s_start(1, s1)

        def body(i, carry):
            j0 = 2 * i
            s_wait(j0, s0)
            s_start(j0 + 2, s0)
            s_wait(j0 + 1, s1)
            s_start(j0 + 3, s1)
            return carry

        lax.fori_loop(0, CH // 2 - 1, body, 0)
        s_wait(CH - 2, s0)
        s_wait(CH - 1, s1)
        plsc.subcore_barrier()
        pltpu.sync_copy(acc_sh.at[pl.ds(sid * RPT, RPT)],
                        out_hbm.at[pl.ds(cid * ACC_N + sid * RPT, RPT)])

    return deg_kernel


def _make_seg_kernel(D):
    @functools.partial(
        pl.kernel,
        mesh=_sc_mesh(),
        compiler_params=pltpu.CompilerParams(use_tc_tiling_on_sc=False),
        out_type=jax.ShapeDtypeStruct((NC * ACC_N, D), jnp.float32),
        scratch_types=[
            pltpu.VMEM((CH, C), jnp.int32),
            pltpu.VMEM((CH, C), jnp.int32),
            pltpu.VMEM((C, D), jnp.float32),
            pltpu.VMEM((C, D), jnp.float32),
            pltpu.VMEM_SHARED((ACC_N, D), jnp.float32),
            pltpu.VMEM_SHARED((ACC_N, D), jnp.float32),
            pltpu.SemaphoreType.DMA,
            pltpu.SemaphoreType.DMA,
            pltpu.SemaphoreType.DMA,
            pltpu.SemaphoreType.DMA,
        ],
    )
    def seg_kernel(y_hbm, src2d, dst2d, zeros_hbm, out_hbm,
                   src_v, dst_v, rows0, rows1, acc_sh, tab_sh, g0, g1, s0, s1):
        cid = lax.axis_index("c")
        sid = lax.axis_index("s")
        wid = cid * NS + sid
        pltpu.sync_copy(src2d.at[pl.ds(wid * CH, CH)], src_v)
        pltpu.sync_copy(dst2d.at[pl.ds(wid * CH, CH)], dst_v)
        pltpu.sync_copy(zeros_hbm.at[pl.ds(sid * RPT, RPT)],
                        acc_sh.at[pl.ds(sid * RPT, RPT)])
        pltpu.sync_copy(y_hbm.at[pl.ds(sid * RPT, RPT)],
                        tab_sh.at[pl.ds(sid * RPT, RPT)])
        plsc.subcore_barrier()

        def g_start(j, buf, sem):
            pltpu.async_copy(tab_sh.at[src_v.at[j]], buf, sem)

        def g_wait(j, buf, sem):
            pltpu.make_async_copy(tab_sh.at[src_v.at[j]], buf, sem).wait()

        def s_start(j, buf, sem):
            pltpu.async_copy(buf, acc_sh.at[dst_v.at[j]], sem, add=True)

        def s_wait(j, buf, sem):
            pltpu.make_async_copy(buf, acc_sh.at[dst_v.at[j]], sem).wait()

        g_start(0, rows0, g0)
        g_start(1, rows1, g1)

        def body(i, carry):
            j0 = 2 * i
            g_wait(j0, rows0, g0)
            s_start(j0, rows0, s0)
            g_wait(j0 + 1, rows1, g1)
            s_start(j0 + 1, rows1, s1)
            s_wait(j0, rows0, s0)
            g_start(j0 + 2, rows0, g0)
            s_wait(j0 + 1, rows1, s1)
            g_start(j0 + 3, rows1, g1)
            return carry

        lax.fori_loop(0, CH // 2 - 1, body, 0)
        j0 = CH - 2
        g_wait(j0, rows0, g0)
        s_start(j0, rows0, s0)
        g_wait(j0 + 1, rows1, g1)
        s_start(j0 + 1, rows1, s1)
        s_wait(j0, rows0, s0)
        s_wait(j0 + 1, rows1, s1)
        plsc.subcore_barrier()
        pltpu.sync_copy(acc_sh.at[pl.ds(sid * RPT, RPT)],
                        out_hbm.at[pl.ds(cid * ACC_N + sid * RPT, RPT)])

    return seg_kernel


_deg_kernel = _make_deg_kernel()
_seg_kernel_h = _make_seg_kernel(D_H)
_seg_kernel_o = _make_seg_kernel(D_OUT)


BN = 1000


def _tc_a_body(x_ref, w_ref, d0_ref, d1_ref, y_ref, dis_ref):
    deg = d0_ref[...] + d1_ref[...] + 1.0
    dis = lax.rsqrt(deg)
    dis_ref[...] = dis
    xw = jnp.dot(x_ref[...], w_ref[...], preferred_element_type=jnp.float32)
    y_ref[...] = dis * xw


def _tc_b_body(a0_ref, a1_ref, y1_ref, dis_ref, w_ref, b_ref, y2_ref):
    dis = dis_ref[...]
    h = dis * (a0_ref[...] + a1_ref[...] + y1_ref[...]) + b_ref[...]
    h = jnp.maximum(h, 0.0)
    y2_ref[...] = dis * jnp.dot(h, w_ref[...], preferred_element_type=jnp.float32)


def _tc_c_body(a0_ref, a1_ref, y2_ref, dis_ref, b_ref, out_ref):
    out_ref[...] = (dis_ref[...] * (a0_ref[...] + a1_ref[...] + y2_ref[...])
                    + b_ref[...])


def _row_spec(d):
    return pl.BlockSpec((BN, d), lambda i: (i, 0))


def _full_spec(shape):
    return pl.BlockSpec(shape, lambda i: (0, 0))


def _tc_a(x, W1, deg0, deg1):
    return pl.pallas_call(
        _tc_a_body,
        grid=(N // BN,),
        in_specs=[_row_spec(D_IN), _full_spec((D_IN, D_H)),
                  _row_spec(1), _row_spec(1)],
        out_specs=[_row_spec(D_H), _row_spec(1)],
        out_shape=[jax.ShapeDtypeStruct((N, D_H), jnp.float32),
                   jax.ShapeDtypeStruct((N, 1), jnp.float32)],
    )(x, W1, deg0, deg1)


def _tc_b(a0, a1, y1, dis, W2, b1):
    return pl.pallas_call(
        _tc_b_body,
        grid=(N // BN,),
        in_specs=[_row_spec(D_H), _row_spec(D_H), _row_spec(D_H), _row_spec(1),
                  _full_spec((D_H, D_OUT)), _full_spec((1, D_H))],
        out_specs=_row_spec(D_OUT),
        out_shape=jax.ShapeDtypeStruct((N, D_OUT), jnp.float32),
    )(a0, a1, y1, dis, W2, b1)


def _tc_c(a0, a1, y2, dis, b2):
    return pl.pallas_call(
        _tc_c_body,
        grid=(N // BN,),
        in_specs=[_row_spec(D_OUT), _row_spec(D_OUT), _row_spec(D_OUT),
                  _row_spec(1), _full_spec((1, D_OUT))],
        out_specs=_row_spec(D_OUT),
        out_shape=jax.ShapeDtypeStruct((N, D_OUT), jnp.float32),
    )(a0, a1, y2, dis, b2)


def kernel(x, edge_index, W1, b1, W2, b2):
    src = edge_index[0]
    dst = edge_index[1]
    pad = E_PAD - E
    sink = jnp.full((pad,), N, dtype=jnp.int32)
    src2d = jnp.concatenate([src, sink]).reshape(NW * CH, C)
    dst2d = jnp.concatenate([dst, sink]).reshape(NW * CH, C)

    ones_rows = jnp.ones((C, DEG_W), jnp.float32)
    zeros64 = jnp.zeros((ACC_N, D_H), jnp.float32)

    degp = _deg_kernel(dst2d, ones_rows, zeros64[:, :DEG_W])
    deg0 = degp[:N, 0:1]
    deg1 = degp[ACC_N:ACC_N + N, 0:1]

    y1, dis = _tc_a(x, W1, deg0, deg1)
    y1_pad = jnp.concatenate([y1, jnp.zeros((ACC_N - N, D_H), jnp.float32)])

    acc1 = _seg_kernel_h(y1_pad, src2d, dst2d, zeros64)
    a10 = acc1[:N]
    a11 = acc1[ACC_N:ACC_N + N]

    y2 = _tc_b(a10, a11, y1, dis, W2, b1.reshape(1, D_H))
    y2_pad = jnp.concatenate([y2, jnp.zeros((ACC_N - N, D_OUT), jnp.float32)])

    acc2 = _seg_kernel_o(y2_pad, src2d, dst2d, zeros64[:, :D_OUT])
    a20 = acc2[:N]
    a21 = acc2[ACC_N:ACC_N + N]

    return _tc_c(a20, a21, y2, dis, b2.reshape(1, D_OUT))


# speedup vs baseline: 38.5872x; 1.1382x over previous
import functools

import jax
import jax.numpy as jnp
from jax import lax
from jax.experimental import pallas as pl
from jax.experimental.pallas import tpu as pltpu
from jax.experimental.pallas import tpu_sc as plsc

N = 10000
E = 320000
D_IN = 128
D_H = 64
D_OUT = 32

NC = 2
NS = 16
NW = NC * NS

C = 125
CH = 80
ROWS = E // C
ACC_N = 10112
RPT = ACC_N // NS
DEG_W = 16

_SC_PARAMS = dict(
    mesh=plsc.VectorSubcoreMesh(core_axis_name="c", subcore_axis_name="s"),
    compiler_params=pltpu.CompilerParams(use_tc_tiling_on_sc=False),
)


def _make_deg_kernel():
    @functools.partial(
        pl.kernel,
        out_type=jax.ShapeDtypeStruct((NC, ACC_N, DEG_W), jnp.float32),
        scratch_types=[
            pltpu.VMEM((CH, C), jnp.int32),
            pltpu.VMEM((C, DEG_W), jnp.float32),
            pltpu.VMEM_SHARED((ACC_N, DEG_W), jnp.float32),
            pltpu.SemaphoreType.DMA,
            pltpu.SemaphoreType.DMA,
        ],
        **_SC_PARAMS,
    )
    def deg_kernel(dst2d, ones_hbm, zeros_hbm, out_hbm, dst_v, ones_v, acc_sh,
                   s0, s1):
        cid = lax.axis_index("c")
        sid = lax.axis_index("s")
        wid = cid * NS + sid
        pltpu.sync_copy(dst2d.at[pl.ds(wid * CH, CH)], dst_v)
        pltpu.sync_copy(ones_hbm, ones_v)
        pltpu.sync_copy(zeros_hbm.at[pl.ds(sid * RPT, RPT)],
                        acc_sh.at[pl.ds(sid * RPT, RPT)])
        plsc.subcore_barrier()

        def s_start(j, sem):
            pltpu.async_copy(ones_v, acc_sh.at[dst_v.at[j]], sem, add=True)

        def s_wait(j, sem):
            pltpu.make_async_copy(ones_v, acc_sh.at[dst_v.at[j]], sem).wait()

        s_start(0, s0)
        s_start(1, s1)

        def body(i, carry):
            j0 = 2 * i
            s_wait(j0, s0)
            s_start(j0 + 2, s0)
            s_wait(j0 + 1, s1)
            s_start(j0 + 3, s1)
            return carry

        lax.fori_loop(0, CH // 2 - 1, body, 0)
        s_wait(CH - 2, s0)
        s_wait(CH - 1, s1)
        plsc.subcore_barrier()
        pltpu.sync_copy(acc_sh.at[pl.ds(sid * RPT, RPT)],
                        out_hbm.at[cid, pl.ds(sid * RPT, RPT)])

    return deg_kernel


def _make_seg_kernel(D):
    @functools.partial(
        pl.kernel,
        out_type=jax.ShapeDtypeStruct((NC, ACC_N, D), jnp.float32),
        scratch_types=[
            pltpu.VMEM((CH, C), jnp.int32),
            pltpu.VMEM((CH, C), jnp.int32),
            pltpu.VMEM((C, D), jnp.float32),
            pltpu.VMEM((C, D), jnp.float32),
            pltpu.VMEM_SHARED((ACC_N, D), jnp.float32),
            pltpu.VMEM_SHARED((ACC_N, D), jnp.float32),
            pltpu.SemaphoreType.DMA,
            pltpu.SemaphoreType.DMA,
            pltpu.SemaphoreType.DMA,
            pltpu.SemaphoreType.DMA,
        ],
        **_SC_PARAMS,
    )
    def seg_kernel(y_hbm, src2d, dst2d, zeros_hbm, out_hbm,
                   src_v, dst_v, rows0, rows1, acc_sh, tab_sh, g0, g1, s0, s1):
        cid = lax.axis_index("c")
        sid = lax.axis_index("s")
        wid = cid * NS + sid
        pltpu.sync_copy(src2d.at[pl.ds(wid * CH, CH)], src_v)
        pltpu.sync_copy(dst2d.at[pl.ds(wid * CH, CH)], dst_v)
        pltpu.sync_copy(zeros_hbm.at[pl.ds(sid * RPT, RPT)],
                        acc_sh.at[pl.ds(sid * RPT, RPT)])
        pltpu.sync_copy(y_hbm.at[pl.ds(sid * RPT, RPT)],
                        tab_sh.at[pl.ds(sid * RPT, RPT)])
        plsc.subcore_barrier()

        def g_start(j, buf, sem):
            pltpu.async_copy(tab_sh.at[src_v.at[j]], buf, sem)

        def g_wait(j, buf, sem):
            pltpu.make_async_copy(tab_sh.at[src_v.at[j]], buf, sem).wait()

        def s_start(j, buf, sem):
            pltpu.async_copy(buf, acc_sh.at[dst_v.at[j]], sem, add=True)

        def s_wait(j, buf, sem):
            pltpu.make_async_copy(buf, acc_sh.at[dst_v.at[j]], sem).wait()

        g_start(0, rows0, g0)
        g_start(1, rows1, g1)

        def body(i, carry):
            j0 = 2 * i
            g_wait(j0, rows0, g0)
            s_start(j0, rows0, s0)
            g_wait(j0 + 1, rows1, g1)
            s_start(j0 + 1, rows1, s1)
            s_wait(j0, rows0, s0)
            g_start(j0 + 2, rows0, g0)
            s_wait(j0 + 1, rows1, s1)
            g_start(j0 + 3, rows1, g1)
            return carry

        lax.fori_loop(0, CH // 2 - 1, body, 0)
        j0 = CH - 2
        g_wait(j0, rows0, g0)
        s_start(j0, rows0, s0)
        g_wait(j0 + 1, rows1, g1)
        s_start(j0 + 1, rows1, s1)
        s_wait(j0, rows0, s0)
        s_wait(j0 + 1, rows1, s1)
        plsc.subcore_barrier()
        pltpu.sync_copy(acc_sh.at[pl.ds(sid * RPT, RPT)],
                        out_hbm.at[cid, pl.ds(sid * RPT, RPT)])

    return seg_kernel


_deg_kernel = _make_deg_kernel()
_seg_kernel_h = _make_seg_kernel(D_H)
_seg_kernel_o = _make_seg_kernel(D_OUT)


BN = 1000


def _mm_body(x_ref, w_ref, o_ref):
    o_ref[...] = jnp.dot(x_ref[...], w_ref[...],
                         preferred_element_type=jnp.float32)


def _scale_body(xw_ref, d0_ref, d1_ref, y_ref, dis_ref):
    deg = d0_ref[0, :, 0:1] + d1_ref[0, :, 0:1] + 1.0
    dis = lax.rsqrt(deg)
    dis_ref[...] = dis
    y_ref[...] = dis * xw_ref[...]


def _tc_b_body(a0_ref, a1_ref, y1_ref, dis_ref, w_ref, b_ref, y2_ref):
    dis = dis_ref[...]
    h = dis * (a0_ref[0] + a1_ref[0] + y1_ref[...]) + b_ref[...]
    h = jnp.maximum(h, 0.0)
    y2_ref[...] = dis * jnp.dot(h, w_ref[...], preferred_element_type=jnp.float32)


def _tc_c_body(a0_ref, a1_ref, y2_ref, dis_ref, b_ref, out_ref):
    out_ref[...] = (dis_ref[...] * (a0_ref[0] + a1_ref[0] + y2_ref[...])
                    + b_ref[...])


def _row_spec(d):
    return pl.BlockSpec((BN, d), lambda i: (i, 0))


def _part_spec(d, c):
    return pl.BlockSpec((1, BN, d), lambda i, c=c: (c, i, 0))


def _full_spec(shape):
    return pl.BlockSpec(shape, lambda i: (0,) * len(shape))


def _tc_mm(x, W1):
    return pl.pallas_call(
        _mm_body,
        grid=(N // BN,),
        in_specs=[_row_spec(D_IN), _full_spec((D_IN, D_H))],
        out_specs=_row_spec(D_H),
        out_shape=jax.ShapeDtypeStruct((N, D_H), jnp.float32),
    )(x, W1)


def _tc_scale(xw, degp):
    return pl.pallas_call(
        _scale_body,
        grid=(N // BN,),
        in_specs=[_row_spec(D_H), _part_spec(DEG_W, 0), _part_spec(DEG_W, 1)],
        out_specs=[_row_spec(D_H), _row_spec(1)],
        out_shape=[jax.ShapeDtypeStruct((ACC_N, D_H), jnp.float32),
                   jax.ShapeDtypeStruct((N, 1), jnp.float32)],
    )(xw, degp, degp)


def _tc_b(acc1, y1, dis, W2, b1):
    return pl.pallas_call(
        _tc_b_body,
        grid=(N // BN,),
        in_specs=[_part_spec(D_H, 0), _part_spec(D_H, 1), _row_spec(D_H),
                  _row_spec(1), _full_spec((D_H, D_OUT)), _full_spec((1, D_H))],
        out_specs=_row_spec(D_OUT),
        out_shape=jax.ShapeDtypeStruct((ACC_N, D_OUT), jnp.float32),
    )(acc1, acc1, y1, dis, W2, b1)


def _tc_c(acc2, y2, dis, b2):
    return pl.pallas_call(
        _tc_c_body,
        grid=(N // BN,),
        in_specs=[_part_spec(D_OUT, 0), _part_spec(D_OUT, 1), _row_spec(D_OUT),
                  _row_spec(1), _full_spec((1, D_OUT))],
        out_specs=_row_spec(D_OUT),
        out_shape=jax.ShapeDtypeStruct((N, D_OUT), jnp.float32),
    )(acc2, acc2, y2, dis, b2)


def kernel(x, edge_index, W1, b1, W2, b2):
    src2d = edge_index[0].reshape(ROWS, C)
    dst2d = edge_index[1].reshape(ROWS, C)

    ones_rows = jnp.ones((C, DEG_W), jnp.float32)
    zeros16 = jnp.zeros((ACC_N, DEG_W), jnp.float32)
    zeros64 = jnp.zeros((ACC_N, D_H), jnp.float32)
    zeros32 = jnp.zeros((ACC_N, D_OUT), jnp.float32)

    degp = _deg_kernel(dst2d, ones_rows, zeros16)
    xw = _tc_mm(x, W1)
    y1, dis = _tc_scale(xw, degp)

    acc1 = _seg_kernel_h(y1, src2d, dst2d, zeros64)
    y2 = _tc_b(acc1, y1, dis, W2, b1.reshape(1, D_H))

    acc2 = _seg_kernel_o(y2, src2d, dst2d, zeros32)
    return _tc_c(acc2, y2, dis, b2.reshape(1, D_OUT))


# speedup vs baseline: 43.4157x; 1.1251x over previous
import functools

import jax
import jax.numpy as jnp
from jax import lax
from jax.experimental import pallas as pl
from jax.experimental.pallas import tpu as pltpu
from jax.experimental.pallas import tpu_sc as plsc

N = 10000
E = 320000
D_IN = 128
D_H = 64
D_OUT = 32

NC = 2
NS = 16
NW = NC * NS

C = 125
CH = 80
ROWS = E // C
ACC_N = 10112
RPT = ACC_N // NS
DEG_W = 16

_SC_PARAMS = dict(
    mesh=plsc.VectorSubcoreMesh(core_axis_name="c", subcore_axis_name="s"),
    compiler_params=pltpu.CompilerParams(use_tc_tiling_on_sc=False),
)


def _make_deg_kernel():
    @functools.partial(
        pl.kernel,
        out_type=jax.ShapeDtypeStruct((ACC_N, 128), jnp.float32),
        scratch_types=[
            pltpu.VMEM((CH, C), jnp.int32),
            pltpu.VMEM((C, DEG_W), jnp.float32),
            pltpu.VMEM_SHARED((ACC_N, DEG_W), jnp.float32),
            pltpu.SemaphoreType.DMA,
            pltpu.SemaphoreType.DMA,
        ],
        **_SC_PARAMS,
    )
    def deg_kernel(dst2d, ones_hbm, zeros_hbm, out_hbm, dst_v, ones_v, acc_sh,
                   s0, s1):
        cid = lax.axis_index("c")
        sid = lax.axis_index("s")
        wid = cid * NS + sid
        pltpu.sync_copy(dst2d.at[pl.ds(wid * CH, CH)], dst_v)
        pltpu.sync_copy(ones_hbm, ones_v)
        pltpu.sync_copy(zeros_hbm.at[pl.ds(sid * RPT, RPT)],
                        acc_sh.at[pl.ds(sid * RPT, RPT)])
        plsc.subcore_barrier()

        def s_start(j, sem):
            pltpu.async_copy(ones_v, acc_sh.at[dst_v.at[j]], sem, add=True)

        def s_wait(j, sem):
            pltpu.make_async_copy(ones_v, acc_sh.at[dst_v.at[j]], sem).wait()

        s_start(0, s0)
        s_start(1, s1)

        def body(i, carry):
            j0 = 2 * i
            s_wait(j0, s0)
            s_start(j0 + 2, s0)
            s_wait(j0 + 1, s1)
            s_start(j0 + 3, s1)
            return carry

        lax.fori_loop(0, CH // 2 - 1, body, 0)
        s_wait(CH - 2, s0)
        s_wait(CH - 1, s1)
        plsc.subcore_barrier()
        pltpu.sync_copy(acc_sh.at[pl.ds(sid * RPT, RPT)],
                        out_hbm.at[pl.ds(sid * RPT, RPT),
                                   pl.ds(cid * DEG_W, DEG_W)])

    return deg_kernel


def _make_seg_kernel(D):
    @functools.partial(
        pl.kernel,
        out_type=jax.ShapeDtypeStruct((ACC_N, 128), jnp.float32),
        scratch_types=[
            pltpu.VMEM((CH, C), jnp.int32),
            pltpu.VMEM((CH, C), jnp.int32),
            pltpu.VMEM((C, D), jnp.float32),
            pltpu.VMEM((C, D), jnp.float32),
            pltpu.VMEM_SHARED((ACC_N, D), jnp.float32),
            pltpu.VMEM_SHARED((ACC_N, D), jnp.float32),
            pltpu.SemaphoreType.DMA,
            pltpu.SemaphoreType.DMA,
            pltpu.SemaphoreType.DMA,
            pltpu.SemaphoreType.DMA,
        ],
        **_SC_PARAMS,
    )
    def seg_kernel(y_hbm, src2d, dst2d, zeros_hbm, out_hbm,
                   src_v, dst_v, rows0, rows1, acc_sh, tab_sh,
                   g0, g1, s0, s1):
        cid = lax.axis_index("c")
        sid = lax.axis_index("s")
        wid = cid * NS + sid
        pltpu.sync_copy(src2d.at[pl.ds(wid * CH, CH)], src_v)
        pltpu.sync_copy(dst2d.at[pl.ds(wid * CH, CH)], dst_v)
        pltpu.sync_copy(zeros_hbm.at[pl.ds(sid * RPT, RPT)],
                        acc_sh.at[pl.ds(sid * RPT, RPT)])
        pltpu.sync_copy(y_hbm.at[pl.ds(sid * RPT, RPT), pl.ds(0, D)],
                        tab_sh.at[pl.ds(sid * RPT, RPT)])
        plsc.subcore_barrier()

        def g_start(j, buf, sem):
            pltpu.async_copy(tab_sh.at[src_v.at[j]], buf, sem)

        def g_wait(j, buf, sem):
            pltpu.make_async_copy(tab_sh.at[src_v.at[j]], buf, sem).wait()

        def s_start(j, buf, sem):
            pltpu.async_copy(buf, acc_sh.at[dst_v.at[j]], sem, add=True)

        def s_wait(j, buf, sem):
            pltpu.make_async_copy(buf, acc_sh.at[dst_v.at[j]], sem).wait()

        g_start(0, rows0, g0)
        g_start(1, rows1, g1)

        def body(i, carry):
            j0 = 2 * i
            g_wait(j0, rows0, g0)
            s_start(j0, rows0, s0)
            g_wait(j0 + 1, rows1, g1)
            s_start(j0 + 1, rows1, s1)
            s_wait(j0, rows0, s0)
            g_start(j0 + 2, rows0, g0)
            s_wait(j0 + 1, rows1, s1)
            g_start(j0 + 3, rows1, g1)
            return carry

        lax.fori_loop(0, CH // 2 - 1, body, 0)
        j0 = CH - 2
        g_wait(j0, rows0, g0)
        s_start(j0, rows0, s0)
        g_wait(j0 + 1, rows1, g1)
        s_start(j0 + 1, rows1, s1)
        s_wait(j0, rows0, s0)
        s_wait(j0 + 1, rows1, s1)
        plsc.subcore_barrier()
        pltpu.sync_copy(acc_sh.at[pl.ds(sid * RPT, RPT)],
                        out_hbm.at[pl.ds(sid * RPT, RPT), pl.ds(cid * D, D)])

    return seg_kernel


_deg_kernel = _make_deg_kernel()
_seg_kernel_h = _make_seg_kernel(D_H)
_seg_kernel_o = _make_seg_kernel(D_OUT)


BN = 1000
G = N // BN


def _mm_body(x_ref, w_ref, o_ref):
    o_ref[...] = jnp.dot(x_ref[...], w_ref[...],
                         preferred_element_type=jnp.float32)


def _scale_body(xw_ref, degc_ref, y_ref, dis_ref):
    d = degc_ref[...]
    deg = d[:, 0:1] + d[:, DEG_W:DEG_W + 1] + 1.0
    dis = lax.rsqrt(deg)
    dis_ref[...] = dis
    y_ref[...] = jnp.concatenate(
        [dis * xw_ref[...], jnp.zeros((BN, 128 - D_H), jnp.float32)], axis=1)


def _tc_b_body(a_ref, y1_ref, dis_ref, w_ref, b_ref, y2_ref):
    dis = dis_ref[...]
    ac = a_ref[...]
    a = ac[:, 0:D_H] + ac[:, D_H:2 * D_H]
    h = dis * (a + y1_ref[:, 0:D_H]) + b_ref[...]
    h = jnp.maximum(h, 0.0)
    y2 = dis * jnp.dot(h, w_ref[...], preferred_element_type=jnp.float32)
    y2_ref[...] = jnp.concatenate(
        [y2, jnp.zeros((BN, 128 - D_OUT), jnp.float32)], axis=1)


def _tc_c_body(a_ref, y2_ref, dis_ref, b_ref, out_ref):
    ac = a_ref[...]
    a = ac[:, 0:D_OUT] + ac[:, D_OUT:2 * D_OUT]
    out_ref[...] = dis_ref[...] * (a + y2_ref[:, 0:D_OUT]) + b_ref[...]


def _row_spec(d):
    return pl.BlockSpec((BN, d), lambda i: (i, 0))


def _full_spec(shape):
    return pl.BlockSpec(shape, lambda i: (0,) * len(shape))


def _tc_mm(x, W1):
    return pl.pallas_call(
        _mm_body,
        grid=(G,),
        in_specs=[_row_spec(D_IN), _full_spec((D_IN, D_H))],
        out_specs=_row_spec(D_H),
        out_shape=jax.ShapeDtypeStruct((N, D_H), jnp.float32),
    )(x, W1)


def _tc_scale(xw, degc):
    return pl.pallas_call(
        _scale_body,
        grid=(G,),
        in_specs=[_row_spec(D_H), _row_spec(128)],
        out_specs=[_row_spec(128), _row_spec(1)],
        out_shape=[jax.ShapeDtypeStruct((ACC_N, 128), jnp.float32),
                   jax.ShapeDtypeStruct((N, 1), jnp.float32)],
    )(xw, degc)


def _tc_b(acc1, y1, dis, W2, b1):
    return pl.pallas_call(
        _tc_b_body,
        grid=(G,),
        in_specs=[_row_spec(128), _row_spec(128), _row_spec(1),
                  _full_spec((D_H, D_OUT)), _full_spec((1, D_H))],
        out_specs=_row_spec(128),
        out_shape=jax.ShapeDtypeStruct((ACC_N, 128), jnp.float32),
    )(acc1, y1, dis, W2, b1)


def _tc_c(acc2, y2, dis, b2):
    return pl.pallas_call(
        _tc_c_body,
        grid=(G,),
        in_specs=[_row_spec(128), _row_spec(128), _row_spec(1),
                  _full_spec((1, D_OUT))],
        out_specs=_row_spec(D_OUT),
        out_shape=jax.ShapeDtypeStruct((N, D_OUT), jnp.float32),
    )(acc2, y2, dis, b2)


def kernel(x, edge_index, W1, b1, W2, b2):
    src2d = edge_index[0].reshape(ROWS, C)
    dst2d = edge_index[1].reshape(ROWS, C)

    ones_rows = jnp.ones((C, DEG_W), jnp.float32)
    zeros16 = jnp.zeros((ACC_N, DEG_W), jnp.float32)
    zeros64 = jnp.zeros((ACC_N, D_H), jnp.float32)
    zeros32 = jnp.zeros((ACC_N, D_OUT), jnp.float32)

    degc = _deg_kernel(dst2d, ones_rows, zeros16)
    xw = _tc_mm(x, W1)
    y1, dis = _tc_scale(xw, degc)

    acc1 = _seg_kernel_h(y1, src2d, dst2d, zeros64)
    y2 = _tc_b(acc1, y1, dis, W2, b1.reshape(1, D_H))

    acc2 = _seg_kernel_o(y2, src2d, dst2d, zeros32)
    return _tc_c(acc2, y2, dis, b2.reshape(1, D_OUT))


# speedup vs baseline: 44.9951x; 1.0364x over previous
import functools

import jax
import jax.numpy as jnp
from jax import lax
from jax.experimental import pallas as pl
from jax.experimental.pallas import tpu as pltpu
from jax.experimental.pallas import tpu_sc as plsc

N = 10000
E = 320000
D_IN = 128
D_H = 64
D_OUT = 32

NC = 2
NS = 16
NW = NC * NS

C = 125
CH = 80
ROWS = E // C
ACC_N = 10112
RPT = ACC_N // NS
DEG_W = 16

_SC_PARAMS = dict(
    mesh=plsc.VectorSubcoreMesh(core_axis_name="c", subcore_axis_name="s"),
    compiler_params=pltpu.CompilerParams(use_tc_tiling_on_sc=False),
)


def _make_deg_kernel():
    @functools.partial(
        pl.kernel,
        out_type=jax.ShapeDtypeStruct((ACC_N, 128), jnp.float32),
        scratch_types=[
            pltpu.VMEM((CH, C), jnp.int32),
            pltpu.VMEM((C, DEG_W), jnp.float32),
            pltpu.VMEM_SHARED((ACC_N, DEG_W), jnp.float32),
            pltpu.SemaphoreType.DMA,
            pltpu.SemaphoreType.DMA,
        ],
        **_SC_PARAMS,
    )
    def deg_kernel(edge3d, ones_hbm, zeros_hbm, out_hbm, dst_v, ones_v, acc_sh,
                   s0, s1):
        cid = lax.axis_index("c")
        sid = lax.axis_index("s")
        wid = cid * NS + sid
        pltpu.sync_copy(edge3d.at[1, pl.ds(wid * CH, CH)], dst_v)
        pltpu.sync_copy(ones_hbm, ones_v)
        pltpu.sync_copy(zeros_hbm.at[pl.ds(sid * RPT, RPT)],
                        acc_sh.at[pl.ds(sid * RPT, RPT)])
        plsc.subcore_barrier()

        def s_start(j, sem):
            pltpu.async_copy(ones_v, acc_sh.at[dst_v.at[j]], sem, add=True)

        def s_wait(j, sem):
            pltpu.make_async_copy(ones_v, acc_sh.at[dst_v.at[j]], sem).wait()

        s_start(0, s0)
        s_start(1, s1)

        def body(i, carry):
            j0 = 2 * i
            s_wait(j0, s0)
            s_start(j0 + 2, s0)
            s_wait(j0 + 1, s1)
            s_start(j0 + 3, s1)
            return carry

        lax.fori_loop(0, CH // 2 - 1, body, 0)
        s_wait(CH - 2, s0)
        s_wait(CH - 1, s1)
        plsc.subcore_barrier()
        pltpu.sync_copy(acc_sh.at[pl.ds(sid * RPT, RPT)],
                        out_hbm.at[pl.ds(sid * RPT, RPT),
                                   pl.ds(cid * DEG_W, DEG_W)])

    return deg_kernel


def _make_seg_kernel(D):
    @functools.partial(
        pl.kernel,
        out_type=jax.ShapeDtypeStruct((ACC_N, 128), jnp.float32),
        scratch_types=[
            pltpu.VMEM((CH, C), jnp.int32),
            pltpu.VMEM((CH, C), jnp.int32),
            pltpu.VMEM((C, D), jnp.float32),
            pltpu.VMEM((C, D), jnp.float32),
            pltpu.VMEM_SHARED((ACC_N, D), jnp.float32),
            pltpu.VMEM_SHARED((ACC_N, D), jnp.float32),
            pltpu.SemaphoreType.DMA,
            pltpu.SemaphoreType.DMA,
            pltpu.SemaphoreType.DMA,
            pltpu.SemaphoreType.DMA,
        ],
        **_SC_PARAMS,
    )
    def seg_kernel(y_hbm, edge3d, zeros_hbm, out_hbm,
                   src_v, dst_v, rows0, rows1, acc_sh, tab_sh,
                   g0, g1, s0, s1):
        cid = lax.axis_index("c")
        sid = lax.axis_index("s")
        wid = cid * NS + sid
        pltpu.sync_copy(edge3d.at[0, pl.ds(wid * CH, CH)], src_v)
        pltpu.sync_copy(edge3d.at[1, pl.ds(wid * CH, CH)], dst_v)
        pltpu.sync_copy(zeros_hbm.at[pl.ds(sid * RPT, RPT)],
                        acc_sh.at[pl.ds(sid * RPT, RPT)])
        pltpu.sync_copy(y_hbm.at[pl.ds(sid * RPT, RPT), pl.ds(0, D)],
                        tab_sh.at[pl.ds(sid * RPT, RPT)])
        plsc.subcore_barrier()

        def g_start(j, buf, sem):
            pltpu.async_copy(tab_sh.at[src_v.at[j]], buf, sem)

        def g_wait(j, buf, sem):
            pltpu.make_async_copy(tab_sh.at[src_v.at[j]], buf, sem).wait()

        def s_start(j, buf, sem):
            pltpu.async_copy(buf, acc_sh.at[dst_v.at[j]], sem, add=True)

        def s_wait(j, buf, sem):
            pltpu.make_async_copy(buf, acc_sh.at[dst_v.at[j]], sem).wait()

        g_start(0, rows0, g0)
        g_start(1, rows1, g1)

        def body(i, carry):
            j0 = 2 * i
            g_wait(j0, rows0, g0)
            s_start(j0, rows0, s0)
            g_wait(j0 + 1, rows1, g1)
            s_start(j0 + 1, rows1, s1)
            s_wait(j0, rows0, s0)
            g_start(j0 + 2, rows0, g0)
            s_wait(j0 + 1, rows1, s1)
            g_start(j0 + 3, rows1, g1)
            return carry

        lax.fori_loop(0, CH // 2 - 1, body, 0)
        j0 = CH - 2
        g_wait(j0, rows0, g0)
        s_start(j0, rows0, s0)
        g_wait(j0 + 1, rows1, g1)
        s_start(j0 + 1, rows1, s1)
        s_wait(j0, rows0, s0)
        s_wait(j0 + 1, rows1, s1)
        plsc.subcore_barrier()
        pltpu.sync_copy(acc_sh.at[pl.ds(sid * RPT, RPT)],
                        out_hbm.at[pl.ds(sid * RPT, RPT), pl.ds(cid * D, D)])

    return seg_kernel


_deg_kernel = _make_deg_kernel()
_seg_kernel_h = _make_seg_kernel(D_H)
_seg_kernel_o = _make_seg_kernel(D_OUT)


BN = 1000
G = N // BN


def _mm_body(x_ref, w_ref, o_ref):
    o_ref[...] = jnp.dot(x_ref[...], w_ref[...],
                         preferred_element_type=jnp.float32)


def _scale_body(xw_ref, degc_ref, y_ref, dis_ref):
    d = degc_ref[...]
    deg = d[:, 0:1] + d[:, DEG_W:DEG_W + 1] + 1.0
    dis = lax.rsqrt(deg)
    dis_ref[...] = dis
    y_ref[...] = jnp.concatenate(
        [dis * xw_ref[...], jnp.zeros((BN, 128 - D_H), jnp.float32)], axis=1)


def _tc_b_body(a_ref, y1_ref, dis_ref, w_ref, b_ref, y2_ref):
    dis = dis_ref[...]
    ac = a_ref[...]
    a = ac[:, 0:D_H] + ac[:, D_H:2 * D_H]
    h = dis * (a + y1_ref[:, 0:D_H]) + b_ref[...]
    h = jnp.maximum(h, 0.0)
    y2 = dis * jnp.dot(h, w_ref[...], preferred_element_type=jnp.float32)
    y2_ref[...] = jnp.concatenate(
        [y2, jnp.zeros((BN, 128 - D_OUT), jnp.float32)], axis=1)


def _tc_c_body(a_ref, y2_ref, dis_ref, b_ref, out_ref):
    ac = a_ref[...]
    a = ac[:, 0:D_OUT] + ac[:, D_OUT:2 * D_OUT]
    out_ref[...] = dis_ref[...] * (a + y2_ref[:, 0:D_OUT]) + b_ref[...]


def _row_spec(d):
    return pl.BlockSpec((BN, d), lambda i: (i, 0))


def _full_spec(shape):
    return pl.BlockSpec(shape, lambda i: (0,) * len(shape))


def _tc_mm(x, W1):
    return pl.pallas_call(
        _mm_body,
        grid=(G,),
        in_specs=[_row_spec(D_IN), _full_spec((D_IN, D_H))],
        out_specs=_row_spec(D_H),
        out_shape=jax.ShapeDtypeStruct((N, D_H), jnp.float32),
    )(x, W1)


def _tc_scale(xw, degc):
    return pl.pallas_call(
        _scale_body,
        grid=(G,),
        in_specs=[_row_spec(D_H), _row_spec(128)],
        out_specs=[_row_spec(128), _row_spec(1)],
        out_shape=[jax.ShapeDtypeStruct((ACC_N, 128), jnp.float32),
                   jax.ShapeDtypeStruct((N, 1), jnp.float32)],
    )(xw, degc)


def _tc_b(acc1, y1, dis, W2, b1):
    return pl.pallas_call(
        _tc_b_body,
        grid=(G,),
        in_specs=[_row_spec(128), _row_spec(128), _row_spec(1),
                  _full_spec((D_H, D_OUT)), _full_spec((1, D_H))],
        out_specs=_row_spec(128),
        out_shape=jax.ShapeDtypeStruct((ACC_N, 128), jnp.float32),
    )(acc1, y1, dis, W2, b1)


def _tc_c(acc2, y2, dis, b2):
    return pl.pallas_call(
        _tc_c_body,
        grid=(G,),
        in_specs=[_row_spec(128), _row_spec(128), _row_spec(1),
                  _full_spec((1, D_OUT))],
        out_specs=_row_spec(D_OUT),
        out_shape=jax.ShapeDtypeStruct((N, D_OUT), jnp.float32),
    )(acc2, y2, dis, b2)


def kernel(x, edge_index, W1, b1, W2, b2):
    edge3d = edge_index.reshape(2, ROWS, C)

    ones_rows = jnp.ones((C, DEG_W), jnp.float32)
    zeros16 = jnp.zeros((ACC_N, DEG_W), jnp.float32)
    zeros64 = jnp.zeros((ACC_N, D_H), jnp.float32)
    zeros32 = jnp.zeros((ACC_N, D_OUT), jnp.float32)

    degc = _deg_kernel(edge3d, ones_rows, zeros16)
    xw = _tc_mm(x, W1)
    y1, dis = _tc_scale(xw, degc)

    acc1 = _seg_kernel_h(y1, edge3d, zeros64)
    y2 = _tc_b(acc1, y1, dis, W2, b1.reshape(1, D_H))

    acc2 = _seg_kernel_o(y2, edge3d, zeros32)
    return _tc_c(acc2, y2, dis, b2.reshape(1, D_OUT))
